# baseline (speedup 1.0000x reference)
import jax
import jax.numpy as jnp
from jax.experimental import pallas as pl
from jax.experimental.pallas import tpu as pltpu

D = 256
NX = NY = NZ = 32
NYZ = NY * NZ
DBLK = 32


def _body(xt_ref, yt_ref, zt_ref, out_ref):
    xt = xt_ref[...]
    yt = yt_ref[...]
    zt = zt_ref[...]
    pos = xt[:, :, None, None] + yt[:, None, :, None] + zt[:, None, None, :]
    out_ref[...] = pos[None]


def kernel(features, x_embed, y_embed, z_embed):
    b = features.shape[0]
    xt = x_embed[:NX].T
    yt = y_embed[:NY].T
    zt = z_embed[:NZ].T
    grid = (b, D // DBLK)
    out = pl.pallas_call(
        _body,
        grid=grid,
        in_specs=[
            pl.BlockSpec((DBLK, NX), lambda bi, di: (di, 0)),
            pl.BlockSpec((DBLK, NY), lambda bi, di: (di, 0)),
            pl.BlockSpec((DBLK, NZ), lambda bi, di: (di, 0)),
        ],
        out_specs=pl.BlockSpec((1, DBLK, NX, NY, NZ),
                               lambda bi, di: (bi, di, 0, 0, 0)),
        out_shape=jax.ShapeDtypeStruct((b, D, NX, NY, NZ), jnp.float32),
    )(xt, yt, zt)
    return out


# speedup vs baseline: 11.2378x; 11.2378x over previous
import jax
import jax.numpy as jnp
from jax.experimental import pallas as pl
from jax.experimental.pallas import tpu as pltpu

D = 256
NX = NY = NZ = 32
XBLK = 8
NSTEP = NX // XBLK
NBUF = 2
B = 4


def _body(xe_ref, ye_ref, ze_ref, out_ref, scratch, sems):
    i = pl.program_id(0)
    slot = jax.lax.rem(i, NBUF)

    xe = xe_ref[...]
    ye = ye_ref[...]
    ze = ze_ref[...]
    yz = ye[:, None, :] + ze[None, :, :]
    pos = xe[:, None, None, :] + yz[None]

    for k in range(NBUF):
        @pl.when(slot == k)
        def _():
            @pl.when(i >= NBUF)
            def _():
                for bb in range(B):
                    pltpu.make_async_copy(
                        scratch.at[k], out_ref.at[bb, pl.ds((i - NBUF) * XBLK, XBLK)],
                        sems.at[k, bb]).wait()

            scratch[k] = pos

            for bb in range(B):
                pltpu.make_async_copy(
                    scratch.at[k], out_ref.at[bb, pl.ds(i * XBLK, XBLK)],
                    sems.at[k, bb]).start()

    @pl.when(i == NSTEP - 1)
    def _():
        for k in range(NBUF):
            step = i - ((i - k) % NBUF)
            for bb in range(B):
                pltpu.make_async_copy(
                    scratch.at[k], out_ref.at[bb, pl.ds(step * XBLK, XBLK)],
                    sems.at[k, bb]).wait()


def kernel(features, x_embed, y_embed, z_embed):
    b = features.shape[0]
    xe = x_embed[:NX]
    ye = y_embed[:NY]
    ze = z_embed[:NZ]
    out = pl.pallas_call(
        _body,
        grid=(NSTEP,),
        in_specs=[
            pl.BlockSpec((XBLK, D), lambda i: (i, 0)),
            pl.BlockSpec((NY, D), lambda i: (0, 0)),
            pl.BlockSpec((NZ, D), lambda i: (0, 0)),
        ],
        out_specs=pl.BlockSpec(memory_space=pl.ANY),
        out_shape=jax.ShapeDtypeStruct((b, NX, NY, NZ, D), jnp.float32),
        scratch_shapes=[
            pltpu.VMEM((NBUF, XBLK, NY, NZ, D), jnp.float32),
            pltpu.SemaphoreType.DMA((NBUF, B)),
        ],
    )(xe, ye, ze)
    return jnp.transpose(out, (0, 4, 1, 2, 3))
